# baseline (speedup 1.0000x reference)
import functools

import jax
import jax.numpy as jnp
from jax import lax
from jax.experimental import pallas as pl
from jax.experimental.pallas import tpu as pltpu
from jax.experimental.pallas import tpu_sc as plsc

_B, _P, _K = 8, 2048, 20
_BH = 4
_NH = _BH * _P
_R = 256
_NRB = _P // _R
_CHUNK = 128
_NCHUNKS = (_K * _NH) // _CHUNK
_NW = 32
_CPW = _NCHUNKS // _NW


def _topk_body(xall_ref, xr_ref, idx_ref):
    b = pl.program_id(0)
    xa = xall_ref[0]
    xr = xr_ref[0]
    dots = lax.dot_general(xa, xr, (((1,), (1,)), ((), ())),
                           preferred_element_type=jnp.float32)
    sqa = jnp.sum(xa * xa, axis=1, keepdims=True)
    xr2 = xr * xr
    ones = jnp.ones((1, xr.shape[1]), jnp.float32)
    sqr = lax.dot_general(ones, xr2, (((1,), (1,)), ((), ())),
                          preferred_element_type=jnp.float32)
    d2 = jnp.maximum(sqa + sqr - 2.0 * dots, 0.0)
    bits = lax.bitcast_convert_type(d2, jnp.int32)
    cand_id = lax.broadcasted_iota(jnp.int32, d2.shape, 0)
    packed = lax.bitwise_or(lax.bitwise_and(bits, jnp.int32(-2048)), cand_id)
    pu = lax.bitcast_convert_type(packed, jnp.uint32)
    m1g = jnp.full((1, _R), 0x80000000, jnp.uint32)
    base = b * _P
    for k in range(_K):
        mn = jnp.min(lax.bitcast_convert_type(pu - m1g, jnp.int32),
                     axis=0, keepdims=True)
        mval = m1g + lax.bitcast_convert_type(mn, jnp.uint32)
        idx_ref[k:k + 1, :] = (
            lax.bitwise_and(lax.bitcast_convert_type(mval, jnp.int32),
                            jnp.int32(2047)) + base)
        m1g = mval + jnp.uint32(0x80000001)


def _topk(x3d, d):
    return pl.pallas_call(
        _topk_body,
        grid=(_BH, _NRB),
        in_specs=[
            pl.BlockSpec((1, _P, d), lambda b, r: (b, 0, 0)),
            pl.BlockSpec((1, _R, d), lambda b, r: (b, r, 0)),
        ],
        out_specs=pl.BlockSpec((_K, _R), lambda b, r: (0, b * _NRB + r)),
        out_shape=jax.ShapeDtypeStruct((_K, _NH), jnp.int32),
    )(x3d, x3d)


@functools.cache
def _sc_gather_fn(d, dtype):
    mesh = plsc.VectorSubcoreMesh(core_axis_name="c", subcore_axis_name="s")

    @functools.partial(
        pl.kernel,
        mesh=mesh,
        out_type=jax.ShapeDtypeStruct((_NCHUNKS, _CHUNK, d), dtype),
        scratch_types=[
            pltpu.VMEM((_CHUNK,), jnp.int32),
            pltpu.VMEM((_CHUNK, d), dtype),
            pltpu.SemaphoreType.DMA,
        ],
    )
    def gather(table_hbm, idx_hbm, out_hbm, idx_v, rows_v, sem):
        wid = lax.axis_index("s") * 2 + lax.axis_index("c")

        def body(t, carry):
            c = wid * _CPW + t
            pltpu.sync_copy(idx_hbm.at[c], idx_v)
            pltpu.async_copy(table_hbm.at[idx_v], rows_v, sem).wait()
            pltpu.sync_copy(rows_v, out_hbm.at[c])
            return carry

        lax.fori_loop(0, _CPW, body, 0)

    return gather


def _sc_gather(table, idx2d):
    return _sc_gather_fn(table.shape[1], table.dtype)(table, idx2d)


def _conv1_body(p0_ref, pj_ref, w1a_ref, b1a_ref, w1b_ref, b1b_ref,
                w2b_ref, w2d_ref, b2_ref, x1_ref, y_ref, z_ref):
    p0 = p0_ref[...]
    w1a = w1a_ref[...]
    w1b = w1b_ref[...]
    b1a = b1a_ref[...]
    acc = None
    for k in range(_K):
        e = p0 + pj_ref[k]
        pre = jnp.maximum(
            jnp.dot(e, w1a, preferred_element_type=jnp.float32) + b1a, 0.0)
        h = jnp.dot(pre, w1b, preferred_element_type=jnp.float32)
        acc = h if acc is None else jnp.maximum(acc, h)
    x1 = acc + b1b_ref[...]
    x1_ref[...] = x1
    y_ref[...] = jnp.dot(x1, w2b_ref[...], preferred_element_type=jnp.float32)
    z_ref[...] = jnp.dot(x1, w2d_ref[...],
                         preferred_element_type=jnp.float32) + b2_ref[...]


def _conv1(p0, pj, w1a128, b1a_r, w1b, b1b_r, w2b, w2d, b2_r):
    t = 512
    return pl.pallas_call(
        _conv1_body,
        grid=(_NH // t,),
        in_specs=[
            pl.BlockSpec((t, 128), lambda i: (i, 0)),
            pl.BlockSpec((_K, t, 128), lambda i: (0, i, 0)),
            pl.BlockSpec((128, 64), lambda i: (0, 0)),
            pl.BlockSpec((1, 64), lambda i: (0, 0)),
            pl.BlockSpec((64, 64), lambda i: (0, 0)),
            pl.BlockSpec((1, 64), lambda i: (0, 0)),
            pl.BlockSpec((64, 128), lambda i: (0, 0)),
            pl.BlockSpec((64, 128), lambda i: (0, 0)),
            pl.BlockSpec((1, 128), lambda i: (0, 0)),
        ],
        out_specs=[
            pl.BlockSpec((t, 64), lambda i: (i, 0)),
            pl.BlockSpec((t, 128), lambda i: (i, 0)),
            pl.BlockSpec((t, 128), lambda i: (i, 0)),
        ],
        out_shape=[
            jax.ShapeDtypeStruct((_NH, 64), jnp.float32),
            jax.ShapeDtypeStruct((_NH, 128), jnp.float32),
            jax.ShapeDtypeStruct((_NH, 128), jnp.float32),
        ],
    )(p0, pj, w1a128, b1a_r, w1b, b1b_r, w2b, w2d, b2_r)


def _final_body(x1_ref, z_ref, yj_ref, wla_ref, wlb_ref, bl_ref, out_ref):
    mx = yj_ref[0]
    for k in range(1, _K):
        mx = jnp.maximum(mx, yj_ref[k])
    x2 = z_ref[...] + mx
    h = (jnp.dot(x1_ref[...], wla_ref[...], preferred_element_type=jnp.float32)
         + jnp.dot(x2, wlb_ref[...], preferred_element_type=jnp.float32)
         + bl_ref[...])
    part = jnp.max(h, axis=0, keepdims=True)

    @pl.when(pl.program_id(1) == 0)
    def _():
        out_ref[0] = part

    @pl.when(pl.program_id(1) != 0)
    def _():
        out_ref[0] = jnp.maximum(out_ref[0], part)


def _final(x1, z, yj, wla, wlb, bl_r):
    t = 512
    npt = _P // t
    return pl.pallas_call(
        _final_body,
        grid=(_BH, npt),
        in_specs=[
            pl.BlockSpec((t, 64), lambda b, i: (b * npt + i, 0)),
            pl.BlockSpec((t, 128), lambda b, i: (b * npt + i, 0)),
            pl.BlockSpec((_K, t, 128), lambda b, i: (0, b * npt + i, 0)),
            pl.BlockSpec((64, 128), lambda b, i: (0, 0)),
            pl.BlockSpec((128, 128), lambda b, i: (0, 0)),
            pl.BlockSpec((1, 128), lambda b, i: (0, 0)),
        ],
        out_specs=pl.BlockSpec((1, 1, 128), lambda b, i: (b, 0, 0)),
        out_shape=jax.ShapeDtypeStruct((_BH, 1, 128), jnp.float32),
    )(x1, z, yj, wla, wlb, bl_r)


def _half(posp, p0, p1, w1a128, b1a_r, W1b, b1b_r, w2b, w2d, b2_r,
          wla, wlb, bl_r):
    idx1 = _topk(posp.reshape(_BH, _P, 8), 8)
    pj = _sc_gather(p1, idx1.reshape(_NCHUNKS, _CHUNK))
    pj = pj.reshape(_K, _NH, 128)
    x1, y, z = _conv1(p0, pj, w1a128, b1a_r, W1b, b1b_r, w2b, w2d, b2_r)

    idx2 = _topk(x1.reshape(_BH, _P, 64), 64)
    yj = _sc_gather(y, idx2.reshape(_NCHUNKS, _CHUNK))
    yj = yj.reshape(_K, _NH, 128)
    return _final(x1, z, yj, wla, wlb, bl_r)


def kernel(pos, batch, W1a, b1a, W1b, b1b, W2, b2, Wl, bl):
    w1a128 = jnp.concatenate([W1a, jnp.zeros((122, 64), jnp.float32)], axis=0)
    w2a, w2b = W2[:64], W2[64:]
    w2d = w2a - w2b
    wla, wlb = Wl[:64], Wl[64:]
    b1a_r = b1a.reshape(1, 64)
    b1b_r = b1b.reshape(1, 64)
    b2_r = b2.reshape(1, 128)
    bl_r = bl.reshape(1, 128)
    n = _B * _P
    posp = jnp.concatenate([pos, jnp.zeros((n, 5), jnp.float32)], axis=1)
    z122 = jnp.zeros((n, 122), jnp.float32)
    p0 = jnp.concatenate([pos, -pos, z122], axis=1)
    p1 = jnp.concatenate([jnp.zeros((n, 3), jnp.float32), pos, z122], axis=1)

    outs = [
        _half(posp[h * _NH:(h + 1) * _NH], p0[h * _NH:(h + 1) * _NH],
              p1[h * _NH:(h + 1) * _NH], w1a128, b1a_r, W1b, b1b_r,
              w2b, w2d, b2_r, wla, wlb, bl_r)
        for h in range(_B // _BH)
    ]
    return jnp.concatenate(outs, axis=0).reshape(_B, 128)


# speedup vs baseline: 1.0298x; 1.0298x over previous
import functools

import jax
import jax.numpy as jnp
from jax import lax
from jax.experimental import pallas as pl
from jax.experimental.pallas import tpu as pltpu
from jax.experimental.pallas import tpu_sc as plsc

_B, _P, _K = 8, 2048, 20
_BH = 2
_NH = _BH * _P
_R = 256
_NRB = _P // _R
_CHUNK = 128
_NCHUNKS = (_K * _NH) // _CHUNK
_NW = 32
_CPW = _NCHUNKS // _NW


def _topk_body(xall_ref, xr_ref, idx_ref):
    b = pl.program_id(0)
    xa = xall_ref[0]
    xr = xr_ref[0]
    dots = lax.dot_general(xa, xr, (((1,), (1,)), ((), ())),
                           preferred_element_type=jnp.float32)
    sqa = jnp.sum(xa * xa, axis=1, keepdims=True)
    xr2 = xr * xr
    ones = jnp.ones((1, xr.shape[1]), jnp.float32)
    sqr = lax.dot_general(ones, xr2, (((1,), (1,)), ((), ())),
                          preferred_element_type=jnp.float32)
    d2 = jnp.maximum(sqa + sqr - 2.0 * dots, 0.0)
    bits = lax.bitcast_convert_type(d2, jnp.int32)
    cand_id = lax.broadcasted_iota(jnp.int32, d2.shape, 0)
    packed = lax.bitwise_or(lax.bitwise_and(bits, jnp.int32(-2048)), cand_id)
    pu = lax.bitcast_convert_type(packed, jnp.uint32)
    m1g = jnp.full((1, _R), 0x80000000, jnp.uint32)
    base = b * _P
    for k in range(_K):
        mn = jnp.min(lax.bitcast_convert_type(pu - m1g, jnp.int32),
                     axis=0, keepdims=True)
        mval = m1g + lax.bitcast_convert_type(mn, jnp.uint32)
        idx_ref[k:k + 1, :] = (
            lax.bitwise_and(lax.bitcast_convert_type(mval, jnp.int32),
                            jnp.int32(2047)) + base)
        m1g = mval + jnp.uint32(0x80000001)


def _topk(x3d, d):
    return pl.pallas_call(
        _topk_body,
        grid=(_BH, _NRB),
        in_specs=[
            pl.BlockSpec((1, _P, d), lambda b, r: (b, 0, 0)),
            pl.BlockSpec((1, _R, d), lambda b, r: (b, r, 0)),
        ],
        out_specs=pl.BlockSpec((_K, _R), lambda b, r: (0, b * _NRB + r)),
        out_shape=jax.ShapeDtypeStruct((_K, _NH), jnp.int32),
    )(x3d, x3d)


@functools.cache
def _sc_gather_fn(d, dtype):
    mesh = plsc.VectorSubcoreMesh(core_axis_name="c", subcore_axis_name="s")

    @functools.partial(
        pl.kernel,
        mesh=mesh,
        out_type=jax.ShapeDtypeStruct((_NCHUNKS, _CHUNK, d), dtype),
        scratch_types=[
            pltpu.VMEM((_CHUNK,), jnp.int32),
            pltpu.VMEM((_CHUNK, d), dtype),
            pltpu.SemaphoreType.DMA,
        ],
    )
    def gather(table_hbm, idx_hbm, out_hbm, idx_v, rows_v, sem):
        wid = lax.axis_index("s") * 2 + lax.axis_index("c")

        def body(t, carry):
            c = wid * _CPW + t
            pltpu.sync_copy(idx_hbm.at[c], idx_v)
            pltpu.async_copy(table_hbm.at[idx_v], rows_v, sem).wait()
            pltpu.sync_copy(rows_v, out_hbm.at[c])
            return carry

        lax.fori_loop(0, _CPW, body, 0)

    return gather


def _sc_gather(table, idx2d):
    return _sc_gather_fn(table.shape[1], table.dtype)(table, idx2d)


def _conv1_body(p0_ref, pj_ref, w1a_ref, b1a_ref, w1b_ref, b1b_ref,
                w2b_ref, w2d_ref, b2_ref, x1_ref, y_ref, z_ref):
    p0 = p0_ref[...]
    w1a = w1a_ref[...]
    w1b = w1b_ref[...]
    b1a = b1a_ref[...]
    acc = None
    for k in range(_K):
        e = p0 + pj_ref[k]
        pre = jnp.maximum(
            jnp.dot(e, w1a, preferred_element_type=jnp.float32) + b1a, 0.0)
        h = jnp.dot(pre, w1b, preferred_element_type=jnp.float32)
        acc = h if acc is None else jnp.maximum(acc, h)
    x1 = acc + b1b_ref[...]
    x1_ref[...] = x1
    y_ref[...] = jnp.dot(x1, w2b_ref[...], preferred_element_type=jnp.float32)
    z_ref[...] = jnp.dot(x1, w2d_ref[...],
                         preferred_element_type=jnp.float32) + b2_ref[...]


def _conv1(p0, pj, w1a128, b1a_r, w1b, b1b_r, w2b, w2d, b2_r):
    t = 512
    return pl.pallas_call(
        _conv1_body,
        grid=(_NH // t,),
        in_specs=[
            pl.BlockSpec((t, 128), lambda i: (i, 0)),
            pl.BlockSpec((_K, t, 128), lambda i: (0, i, 0)),
            pl.BlockSpec((128, 64), lambda i: (0, 0)),
            pl.BlockSpec((1, 64), lambda i: (0, 0)),
            pl.BlockSpec((64, 64), lambda i: (0, 0)),
            pl.BlockSpec((1, 64), lambda i: (0, 0)),
            pl.BlockSpec((64, 128), lambda i: (0, 0)),
            pl.BlockSpec((64, 128), lambda i: (0, 0)),
            pl.BlockSpec((1, 128), lambda i: (0, 0)),
        ],
        out_specs=[
            pl.BlockSpec((t, 64), lambda i: (i, 0)),
            pl.BlockSpec((t, 128), lambda i: (i, 0)),
            pl.BlockSpec((t, 128), lambda i: (i, 0)),
        ],
        out_shape=[
            jax.ShapeDtypeStruct((_NH, 64), jnp.float32),
            jax.ShapeDtypeStruct((_NH, 128), jnp.float32),
            jax.ShapeDtypeStruct((_NH, 128), jnp.float32),
        ],
    )(p0, pj, w1a128, b1a_r, w1b, b1b_r, w2b, w2d, b2_r)


def _final_body(x1_ref, z_ref, yj_ref, wla_ref, wlb_ref, bl_ref, out_ref):
    mx = yj_ref[0]
    for k in range(1, _K):
        mx = jnp.maximum(mx, yj_ref[k])
    x2 = z_ref[...] + mx
    h = (jnp.dot(x1_ref[...], wla_ref[...], preferred_element_type=jnp.float32)
         + jnp.dot(x2, wlb_ref[...], preferred_element_type=jnp.float32)
         + bl_ref[...])
    part = jnp.max(h, axis=0, keepdims=True)

    @pl.when(pl.program_id(1) == 0)
    def _():
        out_ref[0] = part

    @pl.when(pl.program_id(1) != 0)
    def _():
        out_ref[0] = jnp.maximum(out_ref[0], part)


def _final(x1, z, yj, wla, wlb, bl_r):
    t = 512
    npt = _P // t
    return pl.pallas_call(
        _final_body,
        grid=(_BH, npt),
        in_specs=[
            pl.BlockSpec((t, 64), lambda b, i: (b * npt + i, 0)),
            pl.BlockSpec((t, 128), lambda b, i: (b * npt + i, 0)),
            pl.BlockSpec((_K, t, 128), lambda b, i: (0, b * npt + i, 0)),
            pl.BlockSpec((64, 128), lambda b, i: (0, 0)),
            pl.BlockSpec((128, 128), lambda b, i: (0, 0)),
            pl.BlockSpec((1, 128), lambda b, i: (0, 0)),
        ],
        out_specs=pl.BlockSpec((1, 1, 128), lambda b, i: (b, 0, 0)),
        out_shape=jax.ShapeDtypeStruct((_BH, 1, 128), jnp.float32),
    )(x1, z, yj, wla, wlb, bl_r)


def _half(posp, p0, p1, w1a128, b1a_r, W1b, b1b_r, w2b, w2d, b2_r,
          wla, wlb, bl_r):
    idx1 = _topk(posp.reshape(_BH, _P, 8), 8)
    pj = _sc_gather(p1, idx1.reshape(_NCHUNKS, _CHUNK))
    pj = pj.reshape(_K, _NH, 128)
    x1, y, z = _conv1(p0, pj, w1a128, b1a_r, W1b, b1b_r, w2b, w2d, b2_r)

    idx2 = _topk(x1.reshape(_BH, _P, 64), 64)
    yj = _sc_gather(y, idx2.reshape(_NCHUNKS, _CHUNK))
    yj = yj.reshape(_K, _NH, 128)
    return _final(x1, z, yj, wla, wlb, bl_r)


def kernel(pos, batch, W1a, b1a, W1b, b1b, W2, b2, Wl, bl):
    w1a128 = jnp.concatenate([W1a, jnp.zeros((122, 64), jnp.float32)], axis=0)
    w2a, w2b = W2[:64], W2[64:]
    w2d = w2a - w2b
    wla, wlb = Wl[:64], Wl[64:]
    b1a_r = b1a.reshape(1, 64)
    b1b_r = b1b.reshape(1, 64)
    b2_r = b2.reshape(1, 128)
    bl_r = bl.reshape(1, 128)
    n = _B * _P
    posp = jnp.concatenate([pos, jnp.zeros((n, 5), jnp.float32)], axis=1)
    z122 = jnp.zeros((n, 122), jnp.float32)
    p0 = jnp.concatenate([pos, -pos, z122], axis=1)
    p1 = jnp.concatenate([jnp.zeros((n, 3), jnp.float32), pos, z122], axis=1)

    outs = [
        _half(posp[h * _NH:(h + 1) * _NH], p0[h * _NH:(h + 1) * _NH],
              p1[h * _NH:(h + 1) * _NH], w1a128, b1a_r, W1b, b1b_r,
              w2b, w2d, b2_r, wla, wlb, bl_r)
        for h in range(_B // _BH)
    ]
    return jnp.concatenate(outs, axis=0).reshape(_B, 128)
